# baseline (speedup 1.0000x reference)
import functools

import jax
import jax.numpy as jnp
from jax import lax
from jax.experimental import pallas as pl
from jax.experimental.pallas import tpu as pltpu

N = 16384
C = 1000
K = N // 3
R = 2048

_INT_MIN = -2147483648


def _half_stats(pre, lab):
    gap2 = jnp.max(lab, axis=1, keepdims=True)
    colid = lax.broadcasted_iota(jnp.int32, (R // 2, C), 1)
    pseudo2 = jnp.min(jnp.where(lab == gap2, colid, C), axis=1, keepdims=True)
    pick2 = jnp.sum(jnp.where(colid == pseudo2, pre, 0.0), axis=1, keepdims=True)
    m2 = jnp.max(pre, axis=1, keepdims=True)
    s2 = jnp.sum(jnp.exp(pre - m2), axis=1, keepdims=True)
    return gap2, m2 + jnp.log(s2) - pick2


def _stats_kernel(prea_ref, preb_ref, laba_ref, labb_ref, gap_ref, v_ref):
    gap_a, v_a = _half_stats(prea_ref[...], laba_ref[...])
    gap_b, v_b = _half_stats(preb_ref[...], labb_ref[...])
    gap_ref[...] = jnp.concatenate([gap_a, gap_b], axis=0)
    v_ref[...] = jnp.concatenate([v_a, v_b], axis=0)


def _select_kernel(gap_ref, v_ref, out_ref):
    gap = gap_ref[...]
    v = v_ref[...]
    ki = lax.bitcast_convert_type(gap, jnp.int32)
    keys = jnp.where(ki >= 0, ki, ki ^ 0x7FFFFFFF)
    idx = (lax.broadcasted_iota(jnp.int32, (128, 128), 0) * 128
           + lax.broadcasted_iota(jnp.int32, (128, 128), 1))

    def count_ge(t):
        return jnp.sum((keys >= t).astype(jnp.int32))

    t = jnp.where(count_ge(jnp.int32(0)) >= K, jnp.int32(0), jnp.int32(_INT_MIN))
    for b in range(30, -1, -1):
        cand = t + jnp.int32(1 << b)
        t = jnp.where(count_ge(cand) >= K, cand, t)

    tie = keys == t
    need = jnp.int32(K) - jnp.sum((keys > t).astype(jnp.int32))
    lo = jnp.int32(0)
    for b in range(14, -1, -1):
        cand = lo + jnp.int32(1 << b)
        cnt = jnp.sum((tie & (idx < cand)).astype(jnp.int32))
        lo = jnp.where(cnt < need, cand, lo)

    mask = (keys > t) | (tie & (idx <= lo))
    out_ref[0, 0] = jnp.sum(jnp.where(mask, v, 0.0)) / K


@functools.partial(jax.jit, static_argnames=("interpret",))
def kernel(preLogits, Label, interpret=False):
    gap, v = pl.pallas_call(
        _stats_kernel,
        grid=(N // R,),
        in_specs=[
            pl.BlockSpec((R // 2, C), lambda i: (2 * i, 0)),
            pl.BlockSpec((R // 2, C), lambda i: (2 * i + 1, 0)),
            pl.BlockSpec((R // 2, C), lambda i: (2 * i, 0)),
            pl.BlockSpec((R // 2, C), lambda i: (2 * i + 1, 0)),
        ],
        out_specs=[
            pl.BlockSpec((R, 1), lambda i: (i, 0)),
            pl.BlockSpec((R, 1), lambda i: (i, 0)),
        ],
        out_shape=[
            jax.ShapeDtypeStruct((N, 1), jnp.float32),
            jax.ShapeDtypeStruct((N, 1), jnp.float32),
        ],
        compiler_params=pltpu.CompilerParams(
            dimension_semantics=("parallel",)),
        interpret=interpret,
    )(preLogits, preLogits, Label, Label)

    loss = pl.pallas_call(
        _select_kernel,
        in_specs=[
            pl.BlockSpec((128, 128), lambda: (0, 0)),
            pl.BlockSpec((128, 128), lambda: (0, 0)),
        ],
        out_specs=pl.BlockSpec(memory_space=pltpu.SMEM),
        out_shape=jax.ShapeDtypeStruct((1, 1), jnp.float32),
        interpret=interpret,
    )(gap.reshape(128, 128), v.reshape(128, 128))
    return loss[0, 0]


# speedup vs baseline: 1.0020x; 1.0020x over previous
import functools

import jax
import jax.numpy as jnp
from jax import lax
from jax.experimental import pallas as pl
from jax.experimental.pallas import tpu as pltpu

N = 16384
C = 1000
K = N // 3
R = 2048

_INT_MIN = -2147483648


def _stats_kernel(pre_ref, lab_ref, gap_ref, v_ref):
    lab = lab_ref[...]
    pre = pre_ref[...]
    gap2 = jnp.max(lab, axis=1, keepdims=True)
    colid = lax.broadcasted_iota(jnp.int32, (R, C), 1)
    pseudo2 = jnp.min(jnp.where(lab == gap2, colid, C), axis=1, keepdims=True)
    pick2 = jnp.sum(jnp.where(colid == pseudo2, pre, 0.0), axis=1, keepdims=True)
    m2 = jnp.max(pre, axis=1, keepdims=True)
    s2 = jnp.sum(jnp.exp(pre - m2), axis=1, keepdims=True)
    gap_ref[...] = gap2
    v_ref[...] = m2 + jnp.log(s2) - pick2


def _select_kernel(gap_ref, v_ref, out_ref):
    gap = gap_ref[...]
    v = v_ref[...]
    ki = lax.bitcast_convert_type(gap, jnp.int32)
    keys = jnp.where(ki >= 0, ki, ki ^ 0x7FFFFFFF)
    idx = (lax.broadcasted_iota(jnp.int32, (128, 128), 0) * 128
           + lax.broadcasted_iota(jnp.int32, (128, 128), 1))

    def count_ge(t):
        return jnp.sum((keys >= t).astype(jnp.int32))

    t = jnp.where(count_ge(jnp.int32(0)) >= K, jnp.int32(0), jnp.int32(_INT_MIN))
    for b in range(30, -1, -1):
        cand = t + jnp.int32(1 << b)
        t = jnp.where(count_ge(cand) >= K, cand, t)

    tie = keys == t
    need = jnp.int32(K) - jnp.sum((keys > t).astype(jnp.int32))
    lo = jnp.int32(0)
    for b in range(14, -1, -1):
        cand = lo + jnp.int32(1 << b)
        cnt = jnp.sum((tie & (idx < cand)).astype(jnp.int32))
        lo = jnp.where(cnt < need, cand, lo)

    mask = (keys > t) | (tie & (idx <= lo))
    out_ref[0, 0] = jnp.sum(jnp.where(mask, v, 0.0)) / K


@functools.partial(jax.jit, static_argnames=("interpret",))
def kernel(preLogits, Label, interpret=False):
    gap, v = pl.pallas_call(
        _stats_kernel,
        grid=(N // R,),
        in_specs=[
            pl.BlockSpec((R, C), lambda i: (i, 0)),
            pl.BlockSpec((R, C), lambda i: (i, 0)),
        ],
        out_specs=[
            pl.BlockSpec((R, 1), lambda i: (i, 0)),
            pl.BlockSpec((R, 1), lambda i: (i, 0)),
        ],
        out_shape=[
            jax.ShapeDtypeStruct((N, 1), jnp.float32),
            jax.ShapeDtypeStruct((N, 1), jnp.float32),
        ],
        compiler_params=pltpu.CompilerParams(
            dimension_semantics=("parallel",)),
        interpret=interpret,
    )(preLogits, Label)

    loss = pl.pallas_call(
        _select_kernel,
        in_specs=[
            pl.BlockSpec((128, 128), lambda: (0, 0)),
            pl.BlockSpec((128, 128), lambda: (0, 0)),
        ],
        out_specs=pl.BlockSpec(memory_space=pltpu.SMEM),
        out_shape=jax.ShapeDtypeStruct((1, 1), jnp.float32),
        interpret=interpret,
    )(gap.reshape(128, 128), v.reshape(128, 128))
    return loss[0, 0]
